# baseline (speedup 1.0000x reference)
import functools

import jax
import jax.numpy as jnp
from jax import lax
from jax.experimental import pallas as pl
from jax.experimental.pallas import tpu as pltpu
from jax.experimental.pallas import tpu_sc as plsc

_LANES = 16
_NW = 32
_CHUNK = 128
_GPC = _CHUNK // _LANES


@functools.lru_cache(maxsize=None)
def _make_sc_call(per_w: int, n_atoms: int, n_bonds: int):
  n_chunks = per_w // _CHUNK
  n_groups = per_w // _LANES
  mesh = plsc.VectorSubcoreMesh(core_axis_name="c", subcore_axis_name="s")

  @functools.partial(
      pl.kernel,
      mesh=mesh,
      out_type=jax.ShapeDtypeStruct((_NW, _LANES), jnp.float32),
      scratch_types=[
          pltpu.VMEM_SHARED((3 * n_atoms,), jnp.float32),
          pltpu.VMEM((per_w,), jnp.int32),
          pltpu.VMEM((per_w,), jnp.int32),
          pltpu.VMEM((per_w,), jnp.float32),
          pltpu.VMEM((per_w,), jnp.float32),
          pltpu.VMEM((per_w,), jnp.int32),
          pltpu.VMEM((per_w,), jnp.int32),
          pltpu.VMEM((per_w,), jnp.int32),
          pltpu.VMEM((per_w,), jnp.int32),
          pltpu.VMEM((per_w,), jnp.float32),
          pltpu.VMEM((per_w,), jnp.float32),
          pltpu.VMEM((per_w,), jnp.float32),
          pltpu.VMEM((per_w,), jnp.float32),
          pltpu.VMEM((per_w,), jnp.float32),
          pltpu.VMEM((per_w,), jnp.float32),
          pltpu.VMEM((_LANES,), jnp.float32),
          pltpu.SemaphoreType.DMA,
          pltpu.SemaphoreType.DMA,
          pltpu.SemaphoreType.DMA,
      ],
  )
  def sc(cflat_hbm, bflat_hbm, b0_hbm, k_hbm, out_hbm,
         shared_v, ii_v, jj_v, b0_v, k_v,
         yi_i, zi_i, yj_i, zj_i,
         xi_v, yi_v, zi_v, xj_v, yj_v, zj_v,
         acc_v, sem_lin, sem_g, sem_st):
    sid = lax.axis_index("s")
    wid = sid * 2 + lax.axis_index("c")
    wid_start = wid * per_w
    base = jnp.minimum(wid_start, n_bonds - per_w)
    thr = wid_start - base

    @pl.when(sid == 0)
    def _():
      pltpu.async_copy(cflat_hbm, shared_v, sem_st)

    cps = [
        pltpu.async_copy(bflat_hbm.at[pl.ds(base, per_w)], ii_v, sem_lin),
        pltpu.async_copy(bflat_hbm.at[pl.ds(n_bonds + base, per_w)], jj_v,
                         sem_lin),
        pltpu.async_copy(b0_hbm.at[pl.ds(base, per_w)], b0_v, sem_lin),
        pltpu.async_copy(k_hbm.at[pl.ds(base, per_w)], k_v, sem_lin),
    ]
    for cp in cps:
      cp.wait()

    def expand(g, carry):
      s = pl.ds(g * _LANES, _LANES)
      ai = ii_v[s]
      aj = jj_v[s]
      yi_i[s] = ai + n_atoms
      zi_i[s] = ai + 2 * n_atoms
      yj_i[s] = aj + n_atoms
      zj_i[s] = aj + 2 * n_atoms
      return carry

    lax.fori_loop(0, n_groups, expand, 0)

    @pl.when(sid == 0)
    def _():
      pltpu.make_async_copy(cflat_hbm, shared_v, sem_st).wait()

    plsc.subcore_barrier()

    pairs = ((ii_v, xi_v), (yi_i, yi_v), (zi_i, zi_v),
             (jj_v, xj_v), (yj_i, yj_v), (zj_i, zj_v))

    def issue(c, carry):
      s = pl.ds(c * _CHUNK, _CHUNK)
      for idx_ref, dst_ref in pairs:
        pltpu.async_copy(shared_v.at[idx_ref.at[s]], dst_ref.at[s], sem_g)
      return carry

    lax.fori_loop(0, n_chunks, issue, 0)

    lane = lax.iota(jnp.int32, _LANES)

    def group_term(g, acc):
      s = pl.ds(g * _LANES, _LANES)
      dx = xi_v[s] - xj_v[s]
      dy = yi_v[s] - yj_v[s]
      dz = zi_v[s] - zj_v[s]
      d2 = jnp.maximum(dx * dx + dy * dy + dz * dz, jnp.float32(1e-30))
      bits = lax.bitcast_convert_type(d2, jnp.int32)
      est = jnp.int32(0x5F3759DF) - lax.shift_right_arithmetic(bits, 1)
      y = lax.bitcast_convert_type(est, jnp.float32)
      half = jnp.float32(0.5) * d2
      for _ in range(2):
        y = y * (jnp.float32(1.5) - half * y * y)
      dist = d2 * y
      diff = dist - b0_v[s]
      term = k_v[s] * (diff * diff)
      live = (g * _LANES + lane) >= thr
      return acc + jnp.where(live, term, jnp.float32(0.0))

    def chunk_step(c, acc):
      s = pl.ds(c * _CHUNK, _CHUNK)
      for idx_ref, dst_ref in pairs:
        pltpu.make_async_copy(shared_v.at[idx_ref.at[s]], dst_ref.at[s],
                              sem_g).wait()
      for w in range(_GPC):
        acc = group_term(c * _GPC + w, acc)
      return acc

    acc = lax.fori_loop(0, n_chunks, chunk_step,
                        jnp.zeros((_LANES,), jnp.float32))
    acc_v[...] = acc * jnp.float32(0.5)
    pltpu.sync_copy(acc_v, out_hbm.at[wid])

  return sc


def kernel(coords, box, bonds, b0, k_bond):
  del box
  n_bonds = bonds.shape[0]
  n_atoms = coords.shape[0]
  per_w = -(-n_bonds // (_NW * _CHUNK)) * _CHUNK
  cflat = coords.T.reshape(-1)
  bflat = bonds.T.reshape(-1)
  out = _make_sc_call(per_w, n_atoms, n_bonds)(cflat, bflat, b0, k_bond)
  return jnp.sum(out)
